# baseline (speedup 1.0000x reference)
import functools

import jax
import jax.numpy as jnp
from jax import lax
from jax.scipy.linalg import block_diag
from jax.experimental import pallas as pl
from jax.experimental.pallas import tpu as pltpu
from jax.experimental.pallas import tpu_sc as plsc

_NC = 2
_NS = 16
_NW = _NC * _NS
_L = 16


def _silu(v):
    return v * jax.nn.sigmoid(v)


def _node_mlps_body(x_ref, wu1, bu1, wu2, bu2, wv1, bv1, wv2, bv2, hs_ref, hd_ref):
    x = x_ref[...]
    h = _silu(jnp.dot(x, wu1[...], preferred_element_type=jnp.float32) + bu1[...])
    hs_ref[...] = _silu(jnp.dot(h, wu2[...], preferred_element_type=jnp.float32) + bu2[...])
    h = _silu(jnp.dot(x, wv1[...], preferred_element_type=jnp.float32) + bv1[...])
    hd_ref[...] = _silu(jnp.dot(h, wv2[...], preferred_element_type=jnp.float32) + bv2[...])


def _node_mlps(x, wu1, bu1, wu2, bu2, wv1, bv1, wv2, bv2):
    n, d = x.shape
    h = wu1.shape[1]
    bn = 1000
    full = lambda shape: pl.BlockSpec(shape, lambda i: (0,) * len(shape))
    row = pl.BlockSpec((bn, d), lambda i: (i, 0))
    return pl.pallas_call(
        _node_mlps_body,
        grid=(n // bn,),
        in_specs=[row, full((d, h)), full((1, h)), full((h, h)), full((1, h)),
                  full((d, h)), full((1, h)), full((h, h)), full((1, h))],
        out_specs=[pl.BlockSpec((bn, h), lambda i: (i, 0))] * 2,
        out_shape=[jax.ShapeDtypeStruct((n, h), jnp.float32)] * 2,
    )(x, wu1, bu1.reshape(1, h), wu2, bu2.reshape(1, h),
      wv1, bv1.reshape(1, h), wv2, bv2.reshape(1, h))


def _edge_mlp_body(ef_ref, w1, b1, w2, b2, he_ref):
    pack = ef_ref.shape[1] // 16
    h = w2.shape[0]
    h1 = _silu(jnp.dot(ef_ref[...], w1[...], preferred_element_type=jnp.float32) + b1[...])
    for j in range(pack):
        hj = h1[:, h * j:h * (j + 1)]
        he_ref[j] = _silu(jnp.dot(hj, w2[...], preferred_element_type=jnp.float32) + b2[...])


def _edge_mlp(ef128, w1big, b1big, w2, b2):
    e8, dd = ef128.shape
    pack = dd // 16
    h = w2.shape[0]
    be = 400
    full = lambda shape: pl.BlockSpec(shape, lambda i: (0,) * len(shape))
    return pl.pallas_call(
        _edge_mlp_body,
        grid=(e8 // be,),
        in_specs=[pl.BlockSpec((be, dd), lambda i: (i, 0)),
                  full((dd, pack * h)), full((1, pack * h)),
                  full((h, h)), full((1, h))],
        out_specs=pl.BlockSpec((pack, be, h), lambda i: (0, i, 0)),
        out_shape=jax.ShapeDtypeStruct((pack, e8, h), jnp.float32),
    )(ef128, w1big, b1big.reshape(1, pack * h), w2, b2.reshape(1, h))


def _make_sc_deg(n, e):
    epw = e // _NW
    c_sz = 80
    nchunk = epw // c_sz
    rps = (n // _NS) // 8 * 8
    tail = n - rps * _NS
    dw = 128

    mesh = plsc.VectorSubcoreMesh(core_axis_name="c", subcore_axis_name="s")

    @functools.partial(
        pl.kernel,
        out_type=jax.ShapeDtypeStruct((_NC, n, dw), jnp.float32),
        mesh=mesh,
        scratch_types=[
            pltpu.VMEM((c_sz, dw), jnp.float32),
            [pltpu.VMEM((c_sz,), jnp.int32) for _ in range(4)],
            pltpu.VMEM_SHARED((n, dw), jnp.float32),
            pltpu.SemaphoreType.DMA,
            pltpu.SemaphoreType.DMA,
        ],
    )
    def sc_deg(dst_hbm, zerosd_hbm, ones_hbm, order_hbm, deg_hbm, ones_v, idxs,
               deg_sh, isem, ssem):
        del order_hbm
        cid = lax.axis_index("c")
        sid = lax.axis_index("s")
        wid = cid * _NS + sid
        base0 = wid * epw

        pltpu.sync_copy(zerosd_hbm.at[pl.ds(sid * rps, rps)],
                        deg_sh.at[pl.ds(sid * rps, rps)])

        @pl.when(sid == _NS - 1)
        def _():
            pltpu.sync_copy(zerosd_hbm.at[pl.ds(_NS * rps, tail)],
                            deg_sh.at[pl.ds(_NS * rps, tail)])

        pltpu.sync_copy(ones_hbm, ones_v)
        plsc.subcore_barrier()

        def body(q, carry):
            c0 = 4 * q
            for j in range(4):
                pltpu.async_copy(
                    dst_hbm.at[pl.ds(base0 + (c0 + j) * c_sz, c_sz)],
                    idxs[j], isem)
            for j in range(4):
                pltpu.make_async_copy(
                    dst_hbm.at[pl.ds(base0 + (c0 + j) * c_sz, c_sz)],
                    idxs[j], isem).wait()
                pltpu.async_copy(ones_v, deg_sh.at[idxs[j]], ssem, add=True)
            for j in range(4):
                pltpu.make_async_copy(ones_v, deg_sh.at[idxs[j]], ssem).wait()
            return carry

        lax.fori_loop(0, nchunk // 4, body, 0)
        for c in range(nchunk // 4 * 4, nchunk):
            pltpu.sync_copy(dst_hbm.at[pl.ds(base0 + c * c_sz, c_sz)], idxs[0])
            pltpu.async_copy(ones_v, deg_sh.at[idxs[0]], ssem, add=True)
            pltpu.make_async_copy(ones_v, deg_sh.at[idxs[0]], ssem).wait()

        plsc.subcore_barrier()
        pltpu.sync_copy(deg_sh.at[pl.ds(sid * rps, rps)],
                        deg_hbm.at[cid, pl.ds(sid * rps, rps)])

        @pl.when(sid == _NS - 1)
        def _():
            pltpu.sync_copy(deg_sh.at[pl.ds(_NS * rps, tail)],
                            deg_hbm.at[cid, pl.ds(_NS * rps, tail)])

    return sc_deg


def _make_sc_gather_scatter(n, e, h):
    epw = e // _NW
    c_sz = 80
    nchunk = epw // c_sz
    npair = nchunk // 2
    rps = (n // _NS) // 8 * 8
    tail = n - rps * _NS

    mesh = plsc.VectorSubcoreMesh(core_axis_name="c", subcore_axis_name="s")

    nbuf = 4

    @functools.partial(
        pl.kernel,
        out_type=jax.ShapeDtypeStruct((_NC, n, h), jnp.float32),
        mesh=mesh,
        scratch_types=[
            [pltpu.VMEM((c_sz,), jnp.int32) for _ in range(nbuf)],
            [pltpu.VMEM((c_sz,), jnp.int32) for _ in range(nbuf)],
            [pltpu.VMEM((c_sz, h), jnp.float32) for _ in range(nbuf)],
            pltpu.VMEM_SHARED((n, h), jnp.float32),
            [pltpu.SemaphoreType.DMA for _ in range(nbuf)],
            [pltpu.SemaphoreType.DMA for _ in range(nbuf)],
            [pltpu.SemaphoreType.DMA for _ in range(nbuf)],
        ],
    )
    def sc_gs(hn_src_hbm, src_hbm, dst_hbm, zeros_hbm,
              out_hbm,
              sidx, didx, rows, acc, isems, gsems, ssems):
        cid = lax.axis_index("c")
        sid = lax.axis_index("s")
        wid = cid * _NS + sid
        base0 = wid * epw

        pltpu.sync_copy(zeros_hbm.at[pl.ds(sid * rps, rps)],
                        acc.at[pl.ds(sid * rps, rps)])

        @pl.when(sid == _NS - 1)
        def _():
            pltpu.sync_copy(zeros_hbm.at[pl.ds(_NS * rps, tail)],
                            acc.at[pl.ds(_NS * rps, tail)])

        plsc.subcore_barrier()

        def issue_idx(c, j):
            pltpu.async_copy(src_hbm.at[pl.ds(base0 + c * c_sz, c_sz)],
                             sidx[j], isems[j])
            pltpu.async_copy(dst_hbm.at[pl.ds(base0 + c * c_sz, c_sz)],
                             didx[j], isems[j])

        def wait_idx(c, j):
            pltpu.make_async_copy(src_hbm.at[pl.ds(base0 + c * c_sz, c_sz)],
                                  sidx[j], isems[j]).wait()
            pltpu.make_async_copy(dst_hbm.at[pl.ds(base0 + c * c_sz, c_sz)],
                                  didx[j], isems[j]).wait()

        def issue_gather(j):
            pltpu.async_copy(hn_src_hbm.at[sidx[j]], rows[j], gsems[j])

        def wait_gather(j):
            pltpu.make_async_copy(hn_src_hbm.at[sidx[j]], rows[j],
                                  gsems[j]).wait()

        def issue_scatter(j):
            pltpu.async_copy(rows[j], acc.at[didx[j]], ssems[j], add=True)

        def wait_scatter(j):
            pltpu.make_async_copy(rows[j], acc.at[didx[j]], ssems[j]).wait()

        for j in range(nbuf):
            issue_idx(j, j)
        for j in range(nbuf):
            wait_idx(j, j)
            issue_gather(j)

        def quad_body(q, carry):
            c0 = nbuf * q
            for j in range(nbuf):
                wait_gather(j)
                issue_scatter(j)
            for j in range(nbuf):
                c2 = c0 + nbuf + j
                wait_scatter(j)

                @pl.when(c2 < nchunk)
                def _(c2=c2, j=j):
                    issue_idx(c2, j)

            for j in range(nbuf):
                c2 = c0 + nbuf + j

                @pl.when(c2 < nchunk)
                def _(c2=c2, j=j):
                    wait_idx(c2, j)
                    issue_gather(j)

            return carry

        lax.fori_loop(0, nchunk // nbuf, quad_body, 0)

        for j in range(nchunk % nbuf):
            wait_gather(j)
            issue_scatter(j)
            wait_scatter(j)

        plsc.subcore_barrier()
        pltpu.sync_copy(acc.at[pl.ds(sid * rps, rps)],
                        out_hbm.at[cid, pl.ds(sid * rps, rps)])

        @pl.when(sid == _NS - 1)
        def _():
            pltpu.sync_copy(acc.at[pl.ds(_NS * rps, tail)],
                            out_hbm.at[cid, pl.ds(_NS * rps, tail)])

    return sc_gs


def _make_sc_he_scatter(n, e, h):
    epw = e // _NW
    c_sz = 80
    nchunk = epw // c_sz
    npair = nchunk // 2
    rps = (n // _NS) // 8 * 8
    tail = n - rps * _NS

    mesh = plsc.VectorSubcoreMesh(core_axis_name="c", subcore_axis_name="s")

    nbuf = 4

    @functools.partial(
        pl.kernel,
        out_type=jax.ShapeDtypeStruct((_NC, n, h), jnp.float32),
        mesh=mesh,
        scratch_types=[
            [pltpu.VMEM((c_sz,), jnp.int32) for _ in range(nbuf)],
            [pltpu.VMEM((c_sz, h), jnp.float32) for _ in range(nbuf)],
            pltpu.VMEM_SHARED((n, h), jnp.float32),
            [pltpu.SemaphoreType.DMA for _ in range(nbuf)],
            [pltpu.SemaphoreType.DMA for _ in range(nbuf)],
            [pltpu.SemaphoreType.DMA for _ in range(nbuf)],
        ],
    )
    def sc_he(he_hbm, dst_hbm, zeros_hbm, order_hbm,
              out_hbm,
              didx, rows, acc, isems, gsems, ssems):
        del order_hbm
        cid = lax.axis_index("c")
        sid = lax.axis_index("s")
        wid = cid * _NS + sid
        base0 = wid * epw

        pltpu.sync_copy(zeros_hbm.at[pl.ds(sid * rps, rps)],
                        acc.at[pl.ds(sid * rps, rps)])

        @pl.when(sid == _NS - 1)
        def _():
            pltpu.sync_copy(zeros_hbm.at[pl.ds(_NS * rps, tail)],
                            acc.at[pl.ds(_NS * rps, tail)])

        plsc.subcore_barrier()

        def issue_idx(c, j):
            pltpu.async_copy(dst_hbm.at[pl.ds(base0 + c * c_sz, c_sz)],
                             didx[j], isems[j])

        def wait_idx(c, j):
            pltpu.make_async_copy(dst_hbm.at[pl.ds(base0 + c * c_sz, c_sz)],
                                  didx[j], isems[j]).wait()

        def issue_load(c, j):
            pltpu.async_copy(he_hbm.at[pl.ds(base0 + c * c_sz, c_sz)],
                             rows[j], gsems[j])

        def wait_load(c, j):
            pltpu.make_async_copy(he_hbm.at[pl.ds(base0 + c * c_sz, c_sz)],
                                  rows[j], gsems[j]).wait()

        def issue_scatter(j):
            pltpu.async_copy(rows[j], acc.at[didx[j]], ssems[j], add=True)

        def wait_scatter(j):
            pltpu.make_async_copy(rows[j], acc.at[didx[j]], ssems[j]).wait()

        for j in range(nbuf):
            issue_idx(j, j)
            issue_load(j, j)
        for j in range(nbuf):
            wait_idx(j, j)

        def quad_body(q, carry):
            c0 = nbuf * q
            for j in range(nbuf):
                wait_load(c0 + j, j)
                issue_scatter(j)
            for j in range(nbuf):
                c2 = c0 + nbuf + j
                wait_scatter(j)

                @pl.when(c2 < nchunk)
                def _(c2=c2, j=j):
                    issue_idx(c2, j)
                    issue_load(c2, j)

            for j in range(nbuf):
                c2 = c0 + nbuf + j

                @pl.when(c2 < nchunk)
                def _(c2=c2, j=j):
                    wait_idx(c2, j)

            return carry

        lax.fori_loop(0, nchunk // nbuf, quad_body, 0)

        for j in range(nchunk % nbuf):
            cl = nchunk // nbuf * nbuf + j
            wait_load(cl, j)
            issue_scatter(j)
            wait_scatter(j)

        plsc.subcore_barrier()
        pltpu.sync_copy(acc.at[pl.ds(sid * rps, rps)],
                        out_hbm.at[cid, pl.ds(sid * rps, rps)])

        @pl.when(sid == _NS - 1)
        def _():
            pltpu.sync_copy(acc.at[pl.ds(_NS * rps, tail)],
                            out_hbm.at[cid, pl.ds(_NS * rps, tail)])

    return sc_he


def _node_out_body(x_ref, a_ref, b_ref, dg_ref, hd_ref, w1x, w1u, b1, w2, b2,
                   o_ref):
    x = x_ref[...]
    deg = dg_ref[0, :, 0:1] + dg_ref[1, :, 0:1]
    un = (a_ref[0] + a_ref[1]) + (b_ref[0] + b_ref[1]) + deg * hd_ref[...]
    hh = _silu(jnp.dot(x, w1x[...], preferred_element_type=jnp.float32)
               + jnp.dot(un, w1u[...], preferred_element_type=jnp.float32)
               + b1[...])
    o_ref[...] = jnp.dot(hh, w2[...], preferred_element_type=jnp.float32) + b2[...]


def _node_out(x, acc_a, acc_b, deg, hn_dst, wn1, bn1, wn2, bn2):
    n, d = x.shape
    h = wn2.shape[0]
    dw = deg.shape[2]
    bn = 1000
    full = lambda shape: pl.BlockSpec(shape, lambda i: (0,) * len(shape))
    return pl.pallas_call(
        _node_out_body,
        grid=(n // bn,),
        in_specs=[pl.BlockSpec((bn, d), lambda i: (i, 0)),
                  pl.BlockSpec((_NC, bn, h), lambda i: (0, i, 0)),
                  pl.BlockSpec((_NC, bn, h), lambda i: (0, i, 0)),
                  pl.BlockSpec((_NC, bn, dw), lambda i: (0, i, 0)),
                  pl.BlockSpec((bn, h), lambda i: (i, 0)),
                  full((d, h)), full((h, h)), full((1, h)),
                  full((h, h)), full((1, h))],
        out_specs=pl.BlockSpec((bn, h), lambda i: (i, 0)),
        out_shape=jax.ShapeDtypeStruct((n, h), jnp.float32),
    )(x, acc_a, acc_b, deg, hn_dst, wn1[:d], wn1[d:], bn1.reshape(1, h), wn2,
      bn2.reshape(1, h))


def kernel(x, edge_index, edge_feat,
           Wu1, bu1, Wu2, bu2,
           Wv1, bv1, Wv2, bv2,
           We1, be1, We2, be2,
           Wn1, bn1, Wn2, bn2):
    n, d = x.shape
    e = edge_index.shape[1]
    h = Wu1.shape[1]
    pack = 8
    src = edge_index[0].reshape(e // pack, pack).T.reshape(-1)
    dst = edge_index[1].reshape(e // pack, pack).T.reshape(-1)
    dst_p = dst
    hn_src, hn_dst = _node_mlps(x, Wu1, bu1, Wu2, bu2, Wv1, bv1, Wv2, bv2)
    w1big = block_diag(*([We1] * pack))
    b1big = jnp.tile(be1, pack)
    he_packed = _edge_mlp(edge_feat.reshape(e // pack, pack * 16),
                          w1big, b1big, We2, be2)
    he2 = he_packed.reshape(e, h)
    zeros = jnp.zeros((n, h), jnp.float32)
    ones = jnp.ones((80, 128), jnp.float32)
    acc_a = _make_sc_gather_scatter(n, e, h)(hn_src, src, dst, zeros)
    deg = _make_sc_deg(n, e)(dst, zeros, ones, acc_a)
    acc_b = _make_sc_he_scatter(n, e, h)(he2, dst_p, zeros, deg)
    return _node_out(x, acc_a, acc_b, deg, hn_dst, Wn1, bn1, Wn2, bn2)


# speedup vs baseline: 1.1085x; 1.1085x over previous
import functools

import jax
import jax.numpy as jnp
from jax import lax
from jax.scipy.linalg import block_diag
from jax.experimental import pallas as pl
from jax.experimental.pallas import tpu as pltpu
from jax.experimental.pallas import tpu_sc as plsc

_NC = 2
_NS = 16
_NW = _NC * _NS
_L = 16


def _silu(v):
    return v * jax.nn.sigmoid(v)


def _node_mlps_body(x_ref, wu1, bu1, wu2, bu2, wv1, bv1, wv2, bv2, hs_ref, hd_ref):
    x = x_ref[...]
    h = _silu(jnp.dot(x, wu1[...], preferred_element_type=jnp.float32) + bu1[...])
    hs_ref[...] = _silu(jnp.dot(h, wu2[...], preferred_element_type=jnp.float32) + bu2[...])
    h = _silu(jnp.dot(x, wv1[...], preferred_element_type=jnp.float32) + bv1[...])
    hd_ref[...] = _silu(jnp.dot(h, wv2[...], preferred_element_type=jnp.float32) + bv2[...])


def _node_mlps(x, wu1, bu1, wu2, bu2, wv1, bv1, wv2, bv2):
    n, d = x.shape
    h = wu1.shape[1]
    bn = 1000
    full = lambda shape: pl.BlockSpec(shape, lambda i: (0,) * len(shape))
    row = pl.BlockSpec((bn, d), lambda i: (i, 0))
    return pl.pallas_call(
        _node_mlps_body,
        grid=(n // bn,),
        in_specs=[row, full((d, h)), full((1, h)), full((h, h)), full((1, h)),
                  full((d, h)), full((1, h)), full((h, h)), full((1, h))],
        out_specs=[pl.BlockSpec((bn, h), lambda i: (i, 0))] * 2,
        out_shape=[jax.ShapeDtypeStruct((n, h), jnp.float32)] * 2,
    )(x, wu1, bu1.reshape(1, h), wu2, bu2.reshape(1, h),
      wv1, bv1.reshape(1, h), wv2, bv2.reshape(1, h))


def _edge_mlp_body(ef_ref, w1, b1, w2, b2, he_ref):
    pack = ef_ref.shape[1] // 16
    h = w2.shape[0]
    h1 = _silu(jnp.dot(ef_ref[...], w1[...], preferred_element_type=jnp.float32) + b1[...])
    for j in range(pack):
        hj = h1[:, h * j:h * (j + 1)]
        he_ref[j] = _silu(jnp.dot(hj, w2[...], preferred_element_type=jnp.float32) + b2[...])


def _edge_mlp(ef128, w1big, b1big, w2, b2):
    e8, dd = ef128.shape
    pack = dd // 16
    h = w2.shape[0]
    be = 400
    full = lambda shape: pl.BlockSpec(shape, lambda i: (0,) * len(shape))
    return pl.pallas_call(
        _edge_mlp_body,
        grid=(e8 // be,),
        in_specs=[pl.BlockSpec((be, dd), lambda i: (i, 0)),
                  full((dd, pack * h)), full((1, pack * h)),
                  full((h, h)), full((1, h))],
        out_specs=pl.BlockSpec((pack, be, h), lambda i: (0, i, 0)),
        out_shape=jax.ShapeDtypeStruct((pack, e8, h), jnp.float32),
    )(ef128, w1big, b1big.reshape(1, pack * h), w2, b2.reshape(1, h))


def _make_sc_deg(n, e):
    epw = e // _NW
    c_sz = 80
    nchunk = epw // c_sz
    rps = (n // _NS) // 8 * 8
    tail = n - rps * _NS
    dw = 128

    mesh = plsc.VectorSubcoreMesh(core_axis_name="c", subcore_axis_name="s")

    @functools.partial(
        pl.kernel,
        out_type=jax.ShapeDtypeStruct((_NC, n, dw), jnp.float32),
        mesh=mesh,
        scratch_types=[
            pltpu.VMEM((c_sz, dw), jnp.float32),
            [pltpu.VMEM((c_sz,), jnp.int32) for _ in range(4)],
            pltpu.VMEM_SHARED((n, dw), jnp.float32),
            pltpu.SemaphoreType.DMA,
            pltpu.SemaphoreType.DMA,
        ],
    )
    def sc_deg(dst_hbm, zerosd_hbm, ones_hbm, order_hbm, deg_hbm, ones_v, idxs,
               deg_sh, isem, ssem):
        del order_hbm
        cid = lax.axis_index("c")
        sid = lax.axis_index("s")
        wid = cid * _NS + sid
        base0 = wid * epw

        pltpu.sync_copy(zerosd_hbm.at[pl.ds(sid * rps, rps)],
                        deg_sh.at[pl.ds(sid * rps, rps)])

        @pl.when(sid == _NS - 1)
        def _():
            pltpu.sync_copy(zerosd_hbm.at[pl.ds(_NS * rps, tail)],
                            deg_sh.at[pl.ds(_NS * rps, tail)])

        pltpu.sync_copy(ones_hbm, ones_v)
        plsc.subcore_barrier()

        def body(q, carry):
            c0 = 4 * q
            for j in range(4):
                pltpu.async_copy(
                    dst_hbm.at[pl.ds(base0 + (c0 + j) * c_sz, c_sz)],
                    idxs[j], isem)
            for j in range(4):
                pltpu.make_async_copy(
                    dst_hbm.at[pl.ds(base0 + (c0 + j) * c_sz, c_sz)],
                    idxs[j], isem).wait()
                pltpu.async_copy(ones_v, deg_sh.at[idxs[j]], ssem, add=True)
            for j in range(4):
                pltpu.make_async_copy(ones_v, deg_sh.at[idxs[j]], ssem).wait()
            return carry

        lax.fori_loop(0, nchunk // 4, body, 0)
        for c in range(nchunk // 4 * 4, nchunk):
            pltpu.sync_copy(dst_hbm.at[pl.ds(base0 + c * c_sz, c_sz)], idxs[0])
            pltpu.async_copy(ones_v, deg_sh.at[idxs[0]], ssem, add=True)
            pltpu.make_async_copy(ones_v, deg_sh.at[idxs[0]], ssem).wait()

        plsc.subcore_barrier()
        pltpu.sync_copy(deg_sh.at[pl.ds(sid * rps, rps)],
                        deg_hbm.at[cid, pl.ds(sid * rps, rps)])

        @pl.when(sid == _NS - 1)
        def _():
            pltpu.sync_copy(deg_sh.at[pl.ds(_NS * rps, tail)],
                            deg_hbm.at[cid, pl.ds(_NS * rps, tail)])

    return sc_deg


def _make_sc_gather_scatter(n, e, h):
    epw = e // _NW
    c_sz = 80
    nchunk = epw // c_sz
    npair = nchunk // 2
    rps = (n // _NS) // 8 * 8
    tail = n - rps * _NS

    mesh = plsc.VectorSubcoreMesh(core_axis_name="c", subcore_axis_name="s")

    nbuf = 4

    @functools.partial(
        pl.kernel,
        out_type=jax.ShapeDtypeStruct((_NC, n, h), jnp.float32),
        mesh=mesh,
        scratch_types=[
            [pltpu.VMEM((c_sz,), jnp.int32) for _ in range(nbuf)],
            [pltpu.VMEM((c_sz,), jnp.int32) for _ in range(nbuf)],
            [pltpu.VMEM((c_sz, h), jnp.float32) for _ in range(nbuf)],
            pltpu.VMEM_SHARED((n, h), jnp.float32),
            [pltpu.SemaphoreType.DMA for _ in range(nbuf)],
            [pltpu.SemaphoreType.DMA for _ in range(nbuf)],
            [pltpu.SemaphoreType.DMA for _ in range(nbuf)],
        ],
    )
    def sc_gs(hn_src_hbm, src_hbm, dst_hbm, zeros_hbm,
              out_hbm,
              sidx, didx, rows, acc, isems, gsems, ssems):
        cid = lax.axis_index("c")
        sid = lax.axis_index("s")
        wid = cid * _NS + sid
        base0 = wid * epw

        pltpu.sync_copy(zeros_hbm.at[pl.ds(sid * rps, rps)],
                        acc.at[pl.ds(sid * rps, rps)])

        @pl.when(sid == _NS - 1)
        def _():
            pltpu.sync_copy(zeros_hbm.at[pl.ds(_NS * rps, tail)],
                            acc.at[pl.ds(_NS * rps, tail)])

        plsc.subcore_barrier()

        def issue_idx(c, j):
            pltpu.async_copy(src_hbm.at[pl.ds(base0 + c * c_sz, c_sz)],
                             sidx[j], isems[j])
            pltpu.async_copy(dst_hbm.at[pl.ds(base0 + c * c_sz, c_sz)],
                             didx[j], isems[j])

        def wait_idx(c, j):
            pltpu.make_async_copy(src_hbm.at[pl.ds(base0 + c * c_sz, c_sz)],
                                  sidx[j], isems[j]).wait()
            pltpu.make_async_copy(dst_hbm.at[pl.ds(base0 + c * c_sz, c_sz)],
                                  didx[j], isems[j]).wait()

        def issue_gather(j):
            pltpu.async_copy(hn_src_hbm.at[sidx[j]], rows[j], gsems[j])

        def wait_gather(j):
            pltpu.make_async_copy(hn_src_hbm.at[sidx[j]], rows[j],
                                  gsems[j]).wait()

        def issue_scatter(j):
            pltpu.async_copy(rows[j], acc.at[didx[j]], ssems[j], add=True)

        def wait_scatter(j):
            pltpu.make_async_copy(rows[j], acc.at[didx[j]], ssems[j]).wait()

        for j in range(nbuf):
            issue_idx(j, j)
        for j in range(nbuf):
            wait_idx(j, j)
            issue_gather(j)

        def quad_body(q, carry):
            c0 = nbuf * q
            for j in range(nbuf):
                wait_gather(j)
                issue_scatter(j)
            for j in range(nbuf):
                c2 = c0 + nbuf + j
                wait_scatter(j)

                @pl.when(c2 < nchunk)
                def _(c2=c2, j=j):
                    issue_idx(c2, j)

            for j in range(nbuf):
                c2 = c0 + nbuf + j

                @pl.when(c2 < nchunk)
                def _(c2=c2, j=j):
                    wait_idx(c2, j)
                    issue_gather(j)

            return carry

        lax.fori_loop(0, nchunk // nbuf, quad_body, 0)

        for j in range(nchunk % nbuf):
            wait_gather(j)
            issue_scatter(j)
            wait_scatter(j)

        plsc.subcore_barrier()
        pltpu.sync_copy(acc.at[pl.ds(sid * rps, rps)],
                        out_hbm.at[cid, pl.ds(sid * rps, rps)])

        @pl.when(sid == _NS - 1)
        def _():
            pltpu.sync_copy(acc.at[pl.ds(_NS * rps, tail)],
                            out_hbm.at[cid, pl.ds(_NS * rps, tail)])

    return sc_gs


def _make_sc_he_scatter(n, e, h):
    epw = e // _NW
    c_sz = 80
    nchunk = epw // c_sz
    npair = nchunk // 2
    rps = (n // _NS) // 8 * 8
    tail = n - rps * _NS

    mesh = plsc.VectorSubcoreMesh(core_axis_name="c", subcore_axis_name="s")

    nbuf = 4

    @functools.partial(
        pl.kernel,
        out_type=jax.ShapeDtypeStruct((_NC, n, h), jnp.float32),
        mesh=mesh,
        scratch_types=[
            [pltpu.VMEM((c_sz,), jnp.int32) for _ in range(nbuf)],
            [pltpu.VMEM((c_sz, h), jnp.float32) for _ in range(nbuf)],
            pltpu.VMEM_SHARED((n, h), jnp.float32),
            [pltpu.SemaphoreType.DMA for _ in range(nbuf)],
            [pltpu.SemaphoreType.DMA for _ in range(nbuf)],
            [pltpu.SemaphoreType.DMA for _ in range(nbuf)],
        ],
    )
    def sc_he(he_hbm, dst_hbm, zeros_hbm,
              out_hbm,
              didx, rows, acc, isems, gsems, ssems):
        cid = lax.axis_index("c")
        sid = lax.axis_index("s")
        wid = cid * _NS + sid
        base0 = wid * epw

        pltpu.sync_copy(zeros_hbm.at[pl.ds(sid * rps, rps)],
                        acc.at[pl.ds(sid * rps, rps)])

        @pl.when(sid == _NS - 1)
        def _():
            pltpu.sync_copy(zeros_hbm.at[pl.ds(_NS * rps, tail)],
                            acc.at[pl.ds(_NS * rps, tail)])

        plsc.subcore_barrier()

        def issue_idx(c, j):
            pltpu.async_copy(dst_hbm.at[pl.ds(base0 + c * c_sz, c_sz)],
                             didx[j], isems[j])

        def wait_idx(c, j):
            pltpu.make_async_copy(dst_hbm.at[pl.ds(base0 + c * c_sz, c_sz)],
                                  didx[j], isems[j]).wait()

        def issue_load(c, j):
            pltpu.async_copy(he_hbm.at[pl.ds(base0 + c * c_sz, c_sz)],
                             rows[j], gsems[j])

        def wait_load(c, j):
            pltpu.make_async_copy(he_hbm.at[pl.ds(base0 + c * c_sz, c_sz)],
                                  rows[j], gsems[j]).wait()

        def issue_scatter(j):
            pltpu.async_copy(rows[j], acc.at[didx[j]], ssems[j], add=True)

        def wait_scatter(j):
            pltpu.make_async_copy(rows[j], acc.at[didx[j]], ssems[j]).wait()

        for j in range(nbuf):
            issue_idx(j, j)
            issue_load(j, j)
        for j in range(nbuf):
            wait_idx(j, j)

        def quad_body(q, carry):
            c0 = nbuf * q
            for j in range(nbuf):
                wait_load(c0 + j, j)
                issue_scatter(j)
            for j in range(nbuf):
                c2 = c0 + nbuf + j
                wait_scatter(j)

                @pl.when(c2 < nchunk)
                def _(c2=c2, j=j):
                    issue_idx(c2, j)
                    issue_load(c2, j)

            for j in range(nbuf):
                c2 = c0 + nbuf + j

                @pl.when(c2 < nchunk)
                def _(c2=c2, j=j):
                    wait_idx(c2, j)

            return carry

        lax.fori_loop(0, nchunk // nbuf, quad_body, 0)

        for j in range(nchunk % nbuf):
            cl = nchunk // nbuf * nbuf + j
            wait_load(cl, j)
            issue_scatter(j)
            wait_scatter(j)

        plsc.subcore_barrier()
        pltpu.sync_copy(acc.at[pl.ds(sid * rps, rps)],
                        out_hbm.at[cid, pl.ds(sid * rps, rps)])

        @pl.when(sid == _NS - 1)
        def _():
            pltpu.sync_copy(acc.at[pl.ds(_NS * rps, tail)],
                            out_hbm.at[cid, pl.ds(_NS * rps, tail)])

    return sc_he


def _node_out_body(x_ref, a_ref, b_ref, dg_ref, hd_ref, w1x, w1u, b1, w2, b2,
                   o_ref):
    x = x_ref[...]
    deg = dg_ref[0, :, 0:1] + dg_ref[1, :, 0:1]
    un = (a_ref[0] + a_ref[1]) + (b_ref[0] + b_ref[1]) + deg * hd_ref[...]
    hh = _silu(jnp.dot(x, w1x[...], preferred_element_type=jnp.float32)
               + jnp.dot(un, w1u[...], preferred_element_type=jnp.float32)
               + b1[...])
    o_ref[...] = jnp.dot(hh, w2[...], preferred_element_type=jnp.float32) + b2[...]


def _node_out(x, acc_a, acc_b, deg, hn_dst, wn1, bn1, wn2, bn2):
    n, d = x.shape
    h = wn2.shape[0]
    dw = deg.shape[2]
    bn = 1000
    full = lambda shape: pl.BlockSpec(shape, lambda i: (0,) * len(shape))
    return pl.pallas_call(
        _node_out_body,
        grid=(n // bn,),
        in_specs=[pl.BlockSpec((bn, d), lambda i: (i, 0)),
                  pl.BlockSpec((_NC, bn, h), lambda i: (0, i, 0)),
                  pl.BlockSpec((_NC, bn, h), lambda i: (0, i, 0)),
                  pl.BlockSpec((_NC, bn, dw), lambda i: (0, i, 0)),
                  pl.BlockSpec((bn, h), lambda i: (i, 0)),
                  full((d, h)), full((h, h)), full((1, h)),
                  full((h, h)), full((1, h))],
        out_specs=pl.BlockSpec((bn, h), lambda i: (i, 0)),
        out_shape=jax.ShapeDtypeStruct((n, h), jnp.float32),
    )(x, acc_a, acc_b, deg, hn_dst, wn1[:d], wn1[d:], bn1.reshape(1, h), wn2,
      bn2.reshape(1, h))


def kernel(x, edge_index, edge_feat,
           Wu1, bu1, Wu2, bu2,
           Wv1, bv1, Wv2, bv2,
           We1, be1, We2, be2,
           Wn1, bn1, Wn2, bn2):
    n, d = x.shape
    e = edge_index.shape[1]
    h = Wu1.shape[1]
    pack = 8
    src = edge_index[0].reshape(e // pack, pack).T.reshape(-1)
    dst = edge_index[1].reshape(e // pack, pack).T.reshape(-1)
    dst_p = dst
    hn_src, hn_dst = _node_mlps(x, Wu1, bu1, Wu2, bu2, Wv1, bv1, Wv2, bv2)
    w1big = block_diag(*([We1] * pack))
    b1big = jnp.tile(be1, pack)
    he_packed = _edge_mlp(edge_feat.reshape(e // pack, pack * 16),
                          w1big, b1big, We2, be2)
    he2 = he_packed.reshape(e, h)
    zeros = jnp.zeros((n, h), jnp.float32)
    ones = jnp.ones((80, 128), jnp.float32)
    acc_a = _make_sc_gather_scatter(n, e, h)(hn_src, src, dst, zeros)
    deg = _make_sc_deg(n, e)(dst, zeros, ones, acc_a)
    acc_b = _make_sc_he_scatter(n, e, h)(he2, dst_p, zeros)
    return _node_out(x, acc_a, acc_b, deg, hn_dst, Wn1, bn1, Wn2, bn2)
